# baseline (speedup 1.0000x reference)
import functools

import jax
import jax.numpy as jnp
from jax import lax
from jax.experimental import pallas as pl
from jax.experimental.pallas import tpu as pltpu
from jax.experimental.pallas import tpu_sc as plsc

_B = 4096
_L = 200
_D = 64
_N = _B * _L
_NC = 2
_NS = 16
_NW = _NC * _NS
_PW = _N // _NW
_CH = 512
_NCHUNK = _PW // _CH
_LANES = 16


def _lane_sum(vec):
    s = vec[0]
    for i in range(1, _LANES):
        s = s + vec[i]
    return s


def _zero_fixup(idx_v, rows, g_base, flat_base):
    lane = lax.iota(jnp.int32, _LANES)
    zeros = jnp.zeros((_LANES,), jnp.float32)

    def group(r, _):
        start = g_base + r * _LANES
        v = idx_v[pl.ds(start, _LANES)]
        pos = flat_base + start + lane
        m = (v == 0) & (lax.rem(pos, _L) != 0)
        hit = _lane_sum(jnp.where(m, jnp.int32(1), jnp.int32(0)))

        @pl.when(hit > 0)
        def _():
            for j in range(_LANES):
                vj = v[j]
                p = flat_base + start + j

                @pl.when((vj == 0) & (lax.rem(p, _L) != 0))
                def _(j=j):
                    row = r * _LANES + j
                    for k in range(_D // _LANES):
                        rows[row, pl.ds(k * _LANES, _LANES)] = zeros

        return 0

    lax.fori_loop(0, _CH // _LANES, group, 0)


_mesh = plsc.VectorSubcoreMesh(core_axis_name="c", subcore_axis_name="s")


@functools.partial(
    pl.kernel,
    mesh=_mesh,
    out_type=jax.ShapeDtypeStruct((_N, _D), jnp.float32),
    scratch_types=[
        pltpu.VMEM((_PW,), jnp.int32),
        pltpu.VMEM((_CH, _D), jnp.float32),
        pltpu.VMEM((_CH, _D), jnp.float32),
        pltpu.SemaphoreType.DMA,
        pltpu.SemaphoreType.DMA,
    ],
    compiler_params=pltpu.CompilerParams(use_tc_tiling_on_sc=False),
)
def _emb_lookup(idx_hbm, table_hbm, out_hbm, idx_v, rows0, rows1, sem0, sem1):
    wid = lax.axis_index("s") * _NC + lax.axis_index("c")
    base = wid * _PW
    pltpu.sync_copy(idx_hbm.at[pl.ds(base, _PW)], idx_v)

    pltpu.async_copy(table_hbm.at[idx_v.at[pl.ds(0, _CH)]], rows0, sem0)
    pltpu.async_copy(table_hbm.at[idx_v.at[pl.ds(_CH, _CH)]], rows1, sem1)

    def process(g, rows, sem):
        g_base = g * _CH
        pltpu.make_async_copy(
            table_hbm.at[idx_v.at[pl.ds(g_base, _CH)]], rows, sem
        ).wait()

        def red(i, acc):
            v = idx_v[pl.ds(g_base + i * _LANES, _LANES)]
            return acc + jnp.where(v == 0, jnp.int32(1), jnp.int32(0))

        acc = lax.fori_loop(
            0, _CH // _LANES, red, jnp.zeros((_LANES,), jnp.int32)
        )
        nzero = _lane_sum(acc)

        @pl.when(nzero > 0)
        def _():
            _zero_fixup(idx_v, rows, g_base, base)

        pltpu.sync_copy(rows, out_hbm.at[pl.ds(base + g_base, _CH)])

        @pl.when(g + 2 < _NCHUNK)
        def _():
            pltpu.async_copy(
                table_hbm.at[idx_v.at[pl.ds((g + 2) * _CH, _CH)]], rows, sem
            )

    def pair(i, _):
        process(2 * i, rows0, sem0)
        process(2 * i + 1, rows1, sem1)
        return 0

    lax.fori_loop(0, _NCHUNK // 2, pair, 0)


def kernel(x, table):
    idx = x.reshape(_N).astype(jnp.int32)
    out = _emb_lookup(idx, table)
    return out.reshape(_B, _L, _D)
